# baseline (speedup 1.0000x reference)
import functools

import jax
import jax.numpy as jnp
from jax import lax
from jax.experimental import pallas as pl
from jax.experimental.pallas import tpu as pltpu
from jax.experimental.pallas import tpu_sc as plsc

_B = 1024
_L = 200
_D = 64
_CHUNK = 100
_NCHUNKS = _B * _L // _CHUNK
_NC, _NS = 2, 16
_NW = _NC * _NS
_PER_W = _NCHUNKS // _NW


@functools.partial(
    pl.kernel,
    out_type=jax.ShapeDtypeStruct((_NCHUNKS, _CHUNK, _D), jnp.float32),
    mesh=plsc.VectorSubcoreMesh(core_axis_name="c", subcore_axis_name="s",
                                num_cores=_NC),
    scratch_types=[
        pltpu.VMEM((_CHUNK,), jnp.int32),
        pltpu.VMEM((_CHUNK, _D), jnp.float32),
        pltpu.VMEM((_L, _D), jnp.float32),
        pltpu.VMEM_SHARED((_L, _D), jnp.float32),
        pltpu.SemaphoreType.DMA,
    ],
    compiler_params=pltpu.CompilerParams(use_tc_tiling_on_sc=False),
)
def _embed_kernel(ids_hbm, wte_hbm, wpe_hbm, out_hbm, idx_v, rows_v, wpe_v,
                  wpe_sh, sem):
    cid = lax.axis_index("c")
    sid = lax.axis_index("s")
    wid = sid * _NC + cid
    base = wid * _PER_W

    @pl.when(sid == 0)
    def _stage_wpe():
        pltpu.sync_copy(wpe_hbm.at[pl.ds(0, _L)], wpe_v)
        pltpu.sync_copy(wpe_v, wpe_sh)

    plsc.subcore_barrier()

    @pl.loop(0, _PER_W)
    def _chunk(i):
        hr = base + i
        h = lax.rem(i, 2)
        pltpu.sync_copy(ids_hbm.at[hr], idx_v)
        pltpu.sync_copy(wpe_sh.at[pl.ds(h * _CHUNK, _CHUNK)], rows_v)
        pltpu.async_copy(wte_hbm.at[idx_v], rows_v, sem, add=True).wait()
        pltpu.sync_copy(rows_v, out_hbm.at[hr])


def kernel(input_ids, wte_table, wpe_table):
    ids = input_ids.reshape(_NCHUNKS, _CHUNK).astype(jnp.int32)
    out = _embed_kernel(ids, wte_table, wpe_table)
    return out.reshape(_B, _L, _D)


# speedup vs baseline: 1.0763x; 1.0763x over previous
import functools

import jax
import jax.numpy as jnp
from jax import lax
from jax.experimental import pallas as pl
from jax.experimental.pallas import tpu as pltpu
from jax.experimental.pallas import tpu_sc as plsc

_B = 1024
_L = 200
_D = 64
_CHUNK = 100
_CPR = _L // _CHUNK
_NC, _NS = 2, 16
_NW = _NC * _NS
_RPW = _B // _NW
_G = 4


@functools.partial(
    pl.kernel,
    out_type=jax.ShapeDtypeStruct((_B, _CPR, _CHUNK, _D), jnp.float32),
    mesh=plsc.VectorSubcoreMesh(core_axis_name="c", subcore_axis_name="s",
                                num_cores=_NC),
    scratch_types=(
        [pltpu.VMEM((_G, _CPR, _CHUNK), jnp.int32),
         pltpu.VMEM((_G, _CPR, _CHUNK, _D), jnp.float32),
         pltpu.VMEM_SHARED((_CPR, _CHUNK, _D), jnp.float32)]
        + [pltpu.SemaphoreType.DMA] * (4 * _G)
    ),
    compiler_params=pltpu.CompilerParams(use_tc_tiling_on_sc=False),
)
def _embed_kernel(ids_hbm, wte_hbm, wpe_hbm, out_hbm, idx_v, rows_v,
                  wpe_sh, *sems):
    idx_sem = sems[0:_G]
    init_sem = sems[_G:2 * _G]
    g_sem = sems[2 * _G:3 * _G]
    out_sem = sems[3 * _G:4 * _G]

    cid = lax.axis_index("c")
    sid = lax.axis_index("s")
    wid = sid * _NC + cid
    base = wid * _RPW

    @pl.when(sid == 0)
    def _stage_wpe():
        for c in range(_CPR):
            pltpu.sync_copy(wpe_hbm.at[pl.ds(c * _CHUNK, _CHUNK)],
                            rows_v.at[0, c])
            pltpu.sync_copy(rows_v.at[0, c], wpe_sh.at[c])

    plsc.subcore_barrier()

    @pl.loop(0, _RPW, step=_G)
    def _group(g):
        ins = []
        for r in range(_G):
            row = base + g + r
            d_idx = pltpu.async_copy(ids_hbm.at[row], idx_v.at[r],
                                     idx_sem[r])
            d_init = pltpu.async_copy(wpe_sh, rows_v.at[r], init_sem[r])
            ins.append((d_idx, d_init))

        gathers = []
        for r in range(_G):
            ins[r][0].wait()
            ins[r][1].wait()
            for c in range(_CPR):
                gathers.append(
                    pltpu.async_copy(wte_hbm.at[idx_v.at[r, c]],
                                     rows_v.at[r, c], g_sem[r], add=True))

        outs = []
        for r in range(_G):
            for c in range(_CPR):
                gathers[_CPR * r + c].wait()
            outs.append(pltpu.async_copy(rows_v.at[r],
                                         out_hbm.at[base + g + r],
                                         out_sem[r]))

        for d in outs:
            d.wait()


def kernel(input_ids, wte_table, wpe_table):
    ids = input_ids.reshape(_B, _CPR, _CHUNK).astype(jnp.int32)
    out = _embed_kernel(ids, wte_table, wpe_table)
    return out.reshape(_B, _L, _D)
